# baseline (speedup 1.0000x reference)
import functools

import jax
import jax.numpy as jnp
from jax import lax
from jax.experimental import pallas as pl
from jax.experimental.pallas import tpu as pltpu
from jax.experimental.pallas import tpu_sc as plsc

EMB = 64

_info = plsc.get_sparse_core_info()
_NC = _info.num_cores
_NS = _info.num_subcores
_NW = _NC * _NS

CHUNK = 256
NBUF = 4
LAG = 2


def _sc_gather(idx, table):
    B = idx.shape[0]
    assert B % (_NW * CHUNK * NBUF) == 0
    b_per_w = B // _NW
    n_chunks = b_per_w // CHUNK
    n_outer = n_chunks // NBUF
    idx3 = idx.reshape(_NW, n_chunks, CHUNK)

    mesh = plsc.VectorSubcoreMesh(core_axis_name="c", subcore_axis_name="s")

    scratch = (
        [pltpu.VMEM((n_chunks, CHUNK), jnp.int32)]
        + [pltpu.VMEM((CHUNK, EMB), jnp.float32) for _ in range(NBUF)]
        + [pltpu.SemaphoreType.DMA for _ in range(2 * NBUF)]
    )

    @functools.partial(
        pl.kernel,
        mesh=mesh,
        out_type=jax.ShapeDtypeStruct((B, EMB), jnp.float32),
        compiler_params=pltpu.CompilerParams(use_tc_tiling_on_sc=False),
        scratch_types=scratch,
    )
    def k(idx_hbm, table_hbm, out_hbm, idx_v, *bufs_and_sems):
        rows = bufs_and_sems[:NBUF]
        gsem = bufs_and_sems[NBUF : 2 * NBUF]
        osem = bufs_and_sems[2 * NBUF : 3 * NBUF]

        wid = lax.axis_index("s") * _NC + lax.axis_index("c")
        base = wid * b_per_w
        pltpu.sync_copy(idx_hbm.at[wid], idx_v)

        def gather_start(ci, b):
            pltpu.async_copy(table_hbm.at[idx_v.at[ci]], rows[b], gsem[b])

        def gather_wait(ci, b):
            pltpu.make_async_copy(
                table_hbm.at[idx_v.at[ci]], rows[b], gsem[b]
            ).wait()

        def write_start(ci, b):
            pltpu.async_copy(
                rows[b], out_hbm.at[pl.ds(base + ci * CHUNK, CHUNK)], osem[b]
            )

        def write_wait(ci, b):
            pltpu.make_async_copy(
                rows[b], out_hbm.at[pl.ds(base + ci * CHUNK, CHUNK)], osem[b]
            ).wait()

        for b in range(NBUF):
            gather_start(b, b)


        for b in range(NBUF):
            c = b
            gather_wait(c, b)
            write_start(c, b)
            if c >= LAG:
                d = c - LAG
                write_wait(d, d % NBUF)
                gather_start(d + NBUF, d % NBUF)

        def outer(o, carry):
            for b in range(NBUF):
                bd = (b - LAG) % NBUF
                c = o * NBUF + b
                gather_wait(c, b)
                write_start(c, b)
                d = c - LAG
                write_wait(d, bd)
                gather_start(d + NBUF, bd)
            return carry

        lax.fori_loop(1, n_outer - 1, outer, 0)

        for b in range(NBUF):
            c = (n_outer - 1) * NBUF + b
            gather_wait(c, b)
            write_start(c, b)
            d = c - LAG
            write_wait(d, d % NBUF)
            if d + NBUF < n_chunks:
                gather_start(d + NBUF, d % NBUF)
        for c in range(n_chunks - LAG, n_chunks):
            write_wait(c, c % NBUF)

    return k(idx3, table)


def kernel(indices, table):
    idx = indices.reshape(-1).astype(jnp.int32)
    out = _sc_gather(idx, table)
    return out.reshape(indices.shape + (EMB,))
